# baseline (speedup 1.0000x reference)
import jax
import jax.numpy as jnp
from jax import lax
from jax.experimental import pallas as pl
from jax.experimental.pallas import tpu as pltpu
from jax.experimental.pallas import tpu_sc as plsc

N = 10000
NH = N // 2
E = 640000
G = 64
IN = 2
HID = 32
HEADS = 4
EMB = 32
NC = 2

ET = E + N
NCORE = 2
NSUB = 16
NW = NCORE * NSUB
EPT = 20480
EPAD = NW * EPT

CH1 = 4096
NCH1 = EPT // CH1
EQ = EPAD // 4
CH2 = 4096
NCH2 = EQ // CH2

_MESH = plsc.VectorSubcoreMesh(core_axis_name="c", subcore_axis_name="s")
_SC_PARAMS = pltpu.CompilerParams(needs_layout_passes=False)
_HI = lax.Precision.HIGHEST


def _wid():
    return lax.axis_index("c") * NSUB + lax.axis_index("s")


def _zero_ref(ref, n):
    z = jnp.zeros((16,), jnp.float32)

    def body(k, _):
        ref[pl.ds(k * 16, 16)] = z
        return 0

    lax.fori_loop(0, n // 16, body, 0)


def _splat(ref, k):
    return ref[pl.ds(k * 16, 16)]


def _tc_pre_body(w1, a1s, a1d, cs_o, cd_o):
    w1r = w1[...].reshape(IN, HEADS, HID)
    cs_o[...] = jnp.sum(w1r * a1s[...][None], axis=-1)
    cd_o[...] = jnp.sum(w1r * a1d[...][None], axis=-1)


def _tc_pre(w1, a1s, a1d):
    return pl.pallas_call(
        _tc_pre_body,
        out_shape=(jax.ShapeDtypeStruct((IN, HEADS), jnp.float32),
                   jax.ShapeDtypeStruct((IN, HEADS), jnp.float32)),
    )(w1, a1s, a1d)


def _l1_body(x_hbm, cs_hbm, cd_hbm, src_hbm, dst_hbm,
             den_out, num_out,
             x_t, c_t, src_b0, dst_b0, src_b1, dst_b1, den_acc, num_acc,
             sem0, sem1):
    wid = _wid()
    pltpu.sync_copy(x_hbm, x_t)
    pltpu.sync_copy(cs_hbm, c_t.at[pl.ds(0, 128)])
    pltpu.sync_copy(cd_hbm, c_t.at[pl.ds(128, 128)])
    cs = [_splat(c_t, k) for k in range(8)]
    cd = [_splat(c_t, 8 + k) for k in range(8)]
    bufs = ((src_b0, dst_b0, sem0), (src_b1, dst_b1, sem1))

    def issue(c, bi):
        sb, db, sem = bufs[bi]
        base = wid * EPT + c * CH1
        h1 = pltpu.async_copy(src_hbm.at[pl.ds(base, CH1)], sb, sem)
        h2 = pltpu.async_copy(dst_hbm.at[pl.ds(base, CH1)], db, sem)
        return (h1, h2)

    for half in range(2):
        lo = half * NH
        _zero_ref(den_acc, HEADS * NH)
        _zero_ref(num_acc, HEADS * IN * NH)
        hnd = issue(0, 0)
        for c in range(NCH1):
            bi = c % 2
            sb, db, _ = bufs[bi]
            nxt = issue(c + 1, 1 - bi) if c + 1 < NCH1 else None
            hnd[0].wait()
            hnd[1].wait()
            base = wid * EPT + c * CH1

            def step(i, _):
                for u in range(2):
                    o = i * 32 + u * 16
                    s16 = sb[pl.ds(o, 16)]
                    d16 = db[pl.ds(o, 16)]
                    s2 = s16 * 2
                    d2 = d16 * 2
                    x0s = plsc.load_gather(x_t, [s2])
                    x1s = plsc.load_gather(x_t, [s2 + 1])
                    x0d = plsc.load_gather(x_t, [d2])
                    x1d = plsc.load_gather(x_t, [d2 + 1])
                    eid = base + o + lax.iota(jnp.int32, 16)
                    dl = d16 - lo
                    ok = (eid < ET) & (dl >= 0) & (dl < NH)
                    fm = jnp.where(ok, 1.0, 0.0).astype(jnp.float32)
                    di = jnp.where(ok, dl, 0)
                    for h in range(HEADS):
                        e = (x0s * cs[h] + x1s * cs[4 + h]
                             + x0d * cd[h] + x1d * cd[4 + h])
                        e = jnp.where(e > 0, e, 0.2 * e)
                        p = jnp.exp(e) * fm
                        plsc.addupdate_scatter(den_acc, [di + h * NH], p)
                        plsc.addupdate_scatter(
                            num_acc, [di + (h * IN) * NH], p * x0s)
                        plsc.addupdate_scatter(
                            num_acc, [di + (h * IN + 1) * NH], p * x1s)
                return 0

            lax.fori_loop(0, CH1 // 32, step, 0)
            hnd = nxt
        for h in range(HEADS):
            pltpu.sync_copy(
                den_acc.at[pl.ds(h * NH, NH)],
                den_out.at[pl.ds(wid * HEADS * N + h * N + lo, NH)])
        for col in range(HEADS * IN):
            pltpu.sync_copy(
                num_acc.at[pl.ds(col * NH, NH)],
                num_out.at[pl.ds(wid * HEADS * IN * N + col * N + lo, NH)])


_l1 = pl.kernel(
    _l1_body,
    out_type=(jax.ShapeDtypeStruct((NW * HEADS * N,), jnp.float32),
              jax.ShapeDtypeStruct((NW * HEADS * IN * N,), jnp.float32)),
    mesh=_MESH,
    compiler_params=_SC_PARAMS,
    scratch_types=[
        pltpu.VMEM((N * IN,), jnp.float32),
        pltpu.VMEM((256,), jnp.float32),
        pltpu.VMEM((CH1,), jnp.int32),
        pltpu.VMEM((CH1,), jnp.int32),
        pltpu.VMEM((CH1,), jnp.int32),
        pltpu.VMEM((CH1,), jnp.int32),
        pltpu.VMEM((HEADS * NH,), jnp.float32),
        pltpu.VMEM((HEADS * IN * NH,), jnp.float32),
        pltpu.SemaphoreType.DMA,
        pltpu.SemaphoreType.DMA,
    ],
)


def _l2_body(h2_hbm, as_hbm, ad_hbm, src_hbm, dst_hbm,
             num_out, den_out,
             h2_t, as_t, ad_t, src_b0, dst_b0, src_b1, dst_b1,
             num_acc, den_acc, sem0, sem1):
    wid = _wid()
    g = wid % 8
    q = wid // 8
    pltpu.sync_copy(h2_hbm.at[pl.ds(g * 4 * N, 4 * N)], h2_t)
    pltpu.sync_copy(as_hbm, as_t)
    pltpu.sync_copy(ad_hbm, ad_t)
    _zero_ref(num_acc, 4 * N)
    _zero_ref(den_acc, N)
    gz = jnp.where(g == 0, 1.0, 0.0).astype(jnp.float32)
    gzv = jnp.zeros((16,), jnp.float32) + gz
    bufs = ((src_b0, dst_b0, sem0), (src_b1, dst_b1, sem1))

    def issue(c, bi):
        sb, db, sem = bufs[bi]
        base = q * EQ + c * CH2
        h1 = pltpu.async_copy(src_hbm.at[pl.ds(base, CH2)], sb, sem)
        h2 = pltpu.async_copy(dst_hbm.at[pl.ds(base, CH2)], db, sem)
        return (h1, h2)

    hnd = issue(0, 0)
    for c in range(NCH2):
        bi = c % 2
        sb, db, _ = bufs[bi]
        nxt = issue(c + 1, 1 - bi) if c + 1 < NCH2 else None
        hnd[0].wait()
        hnd[1].wait()
        base = q * EQ + c * CH2

        def step(i, _):
            for u in range(2):
                o = i * 32 + u * 16
                s16 = sb[pl.ds(o, 16)]
                d16 = db[pl.ds(o, 16)]
                av = plsc.load_gather(as_t, [s16])
                bv = plsc.load_gather(ad_t, [d16])
                e = av + bv
                e = jnp.where(e > 0, e, 0.2 * e)
                eid = base + o + lax.iota(jnp.int32, 16)
                fm = jnp.where(eid < ET, 1.0, 0.0).astype(jnp.float32)
                p = jnp.exp(e) * fm
                plsc.addupdate_scatter(den_acc, [d16], p * gzv)
                for j in range(4):
                    hv = plsc.load_gather(h2_t, [s16 + j * N])
                    plsc.addupdate_scatter(num_acc, [d16 + j * N], p * hv)
            return 0

        lax.fori_loop(0, CH2 // 32, step, 0)
        hnd = nxt

    pltpu.sync_copy(num_acc, num_out.at[pl.ds(wid * 4 * N, 4 * N)])

    @pl.when(g == 0)
    def _():
        pltpu.sync_copy(den_acc, den_out.at[pl.ds(q * N, N)])


_l2 = pl.kernel(
    _l2_body,
    out_type=(jax.ShapeDtypeStruct((NW * 4 * N,), jnp.float32),
              jax.ShapeDtypeStruct((4 * N,), jnp.float32)),
    mesh=_MESH,
    compiler_params=_SC_PARAMS,
    scratch_types=[
        pltpu.VMEM((4 * N,), jnp.float32),
        pltpu.VMEM((N,), jnp.float32),
        pltpu.VMEM((N,), jnp.float32),
        pltpu.VMEM((CH2,), jnp.int32),
        pltpu.VMEM((CH2,), jnp.int32),
        pltpu.VMEM((CH2,), jnp.int32),
        pltpu.VMEM((CH2,), jnp.int32),
        pltpu.VMEM((4 * N,), jnp.float32),
        pltpu.VMEM((N,), jnp.float32),
        pltpu.SemaphoreType.DMA,
        pltpu.SemaphoreType.DMA,
    ],
)


def _tc_mid_body(den_p, num_p, w1t, b1, g1, be1, w2, as2w, ad2w,
                 h2t_o, as2_o, ad2_o):
    den = jnp.sum(den_p[...], axis=0)
    num = jnp.sum(num_p[...], axis=0)
    den_r = jnp.broadcast_to(den.reshape(HEADS, 1, N),
                             (HEADS, IN, N)).reshape(HEADS * IN, N)
    qn = num / (den_r + 1e-16)
    wt = jnp.concatenate([w1t[...]] * HEADS, axis=1)
    r = lax.broadcasted_iota(jnp.int32, (HEADS * HID, HEADS * IN), 0)
    c = lax.broadcasted_iota(jnp.int32, (HEADS * HID, HEADS * IN), 1)
    bd = jnp.where(r // HID == c // IN, wt, 0.0)
    h = jnp.dot(bd, qn, preferred_element_type=jnp.float32,
                precision=_HI) + b1[...]
    mu = jnp.mean(h, axis=1, keepdims=True)
    var = jnp.mean((h - mu) ** 2, axis=1, keepdims=True)
    h = (h - mu) / jnp.sqrt(var + 1e-5) * g1[...] + be1[...]
    h = jnp.where(h > 0, h, jnp.exp(jnp.minimum(h, 0.0)) - 1.0)
    h2 = lax.dot_general(w2[...], h, (((0,), (0,)), ((), ())),
                         preferred_element_type=jnp.float32,
                         precision=_HI)
    h2t_o[...] = h2
    as2_o[...] = jnp.dot(as2w[...], h2, preferred_element_type=jnp.float32,
                         precision=_HI)
    ad2_o[...] = jnp.dot(ad2w[...], h2, preferred_element_type=jnp.float32,
                         precision=_HI)


def _tc_mid(den_p, num_p, w1t, b1, g1, be1, w2, as2w, ad2w):
    return pl.pallas_call(
        _tc_mid_body,
        out_shape=(jax.ShapeDtypeStruct((EMB, N), jnp.float32),
                   jax.ShapeDtypeStruct((1, N), jnp.float32),
                   jax.ShapeDtypeStruct((1, N), jnp.float32)),
    )(den_p, num_p, w1t, b1, g1, be1, w2, as2w, ad2w)


def _tc_out_body(den_p, num_p, batch, b2, g2, be2, wc, bc, out):
    den = jnp.sum(den_p[...], axis=0).reshape(1, N)
    num = jnp.sum(num_p[...].reshape(4, 8, 4, N), axis=0).reshape(EMB, N)
    h = num / (den + 1e-16) + b2[...]
    mu = jnp.mean(h, axis=1, keepdims=True)
    var = jnp.mean((h - mu) ** 2, axis=1, keepdims=True)
    h = (h - mu) / jnp.sqrt(var + 1e-5) * g2[...] + be2[...]
    h = jnp.where(h > 0, h, jnp.exp(jnp.minimum(h, 0.0)) - 1.0)
    seg = (batch[...] == lax.broadcasted_iota(jnp.int32, (N, G), 1))
    seg = seg.astype(jnp.float32)
    s = jnp.dot(h, seg, preferred_element_type=jnp.float32,
                precision=_HI)
    cnt = jnp.sum(seg, axis=0, keepdims=True)
    emb = s / jnp.maximum(cnt, 1.0)
    out[...] = lax.dot_general(emb, wc[...], (((0,), (0,)), ((), ())),
                               preferred_element_type=jnp.float32,
                               precision=_HI) + bc[...]


def _tc_out(den_p, num_p, batch, b2, g2, be2, wc, bc):
    return pl.pallas_call(
        _tc_out_body,
        out_shape=jax.ShapeDtypeStruct((G, NC), jnp.float32),
    )(den_p, num_p, batch, b2, g2, be2, wc, bc)


def kernel(x, edge_index, batch, W1, a_src1, a_dst1, b1, g1, be1,
           W2, a_src2, a_dst2, b2, g2, be2, Wc, bc):
    loop = jnp.arange(N, dtype=jnp.int32)
    padi = jnp.zeros((EPAD - ET,), jnp.int32)
    src = jnp.concatenate([edge_index[0], loop, padi])
    dst = jnp.concatenate([edge_index[1], loop, padi])

    cs, cd = _tc_pre(W1, a_src1, a_dst1)
    csp = jnp.broadcast_to(cs.reshape(-1)[:, None], (8, 16)).reshape(-1)
    cdp = jnp.broadcast_to(cd.reshape(-1)[:, None], (8, 16)).reshape(-1)
    den1p, num1p = _l1(x.reshape(-1), csp, cdp, src, dst)
    h2t, as2, ad2 = _tc_mid(den1p.reshape(NW, HEADS, N),
                            num1p.reshape(NW, HEADS * IN, N),
                            W1.T, b1.reshape(-1, 1), g1.reshape(-1, 1),
                            be1.reshape(-1, 1), W2, a_src2, a_dst2)
    num2p, den2p = _l2(h2t.reshape(-1), as2.reshape(-1), ad2.reshape(-1),
                       src, dst)
    out = _tc_out(den2p.reshape(4, N), num2p.reshape(NW, 4, N),
                  batch.reshape(-1, 1), b2.reshape(-1, 1),
                  g2.reshape(-1, 1), be2.reshape(-1, 1), Wc,
                  bc.reshape(1, -1))
    return out
